# baseline (speedup 1.0000x reference)
import functools

import jax
import jax.numpy as jnp
from jax import lax
from jax.experimental import pallas as pl
from jax.experimental.pallas import tpu as pltpu
from jax.experimental.pallas import tpu_sc as plsc

BATCH = 16384
N_FIELDS = 26
EMBED_DIM = 16

NC, NS, LANES = 2, 16, 16
NW = NC * NS
B_PER_W = BATCH // NW
IDX_PER_W = B_PER_W * N_FIELDS

FEAT = 1000000
PER_SUB = 62496
SCH = 10416
N_SCH = PER_SUB // SCH
REM = FEAT - NS * PER_SUB


def _first_order_sc(idx_f, w_row):
    mesh = plsc.VectorSubcoreMesh(core_axis_name="c", subcore_axis_name="s")

    @functools.partial(
        pl.kernel,
        out_type=jax.ShapeDtypeStruct((1, BATCH), jnp.float32),
        mesh=mesh,
        scratch_types=[
            pltpu.VMEM((IDX_PER_W,), jnp.int32),
            pltpu.VMEM((IDX_PER_W,), jnp.float32),
            pltpu.VMEM((B_PER_W,), jnp.float32),
            pltpu.VMEM((SCH,), jnp.float32),
            pltpu.VMEM((SCH,), jnp.float32),
            pltpu.VMEM_SHARED((FEAT,), jnp.float32),
            pltpu.SemaphoreType.DMA,
            pltpu.SemaphoreType.DMA,
            pltpu.SemaphoreType.DMA,
            pltpu.SemaphoreType.DMA,
            pltpu.SemaphoreType.DMA,
            pltpu.SemaphoreType.DMA,
        ],
    )
    def k(idx_hbm, w_hbm, out_hbm, idx_v, vals_v, out_v, buf_a, buf_b,
          w_sh, sem, la, lb, sa, sb, isem):
        sid = lax.axis_index("s")
        wid = sid * NC + lax.axis_index("c")
        idx_copies = [
            pltpu.make_async_copy(
                idx_hbm.at[pl.ds(f * BATCH + wid * B_PER_W, B_PER_W)],
                idx_v.at[pl.ds(f * B_PER_W, B_PER_W)], isem)
            for f in range(N_FIELDS)
        ]
        for c in idx_copies:
            c.start()

        bufs = (buf_a, buf_b)
        lsems = (la, lb)
        ssems = (sa, sb)
        loads = [
            pltpu.make_async_copy(
                w_hbm.at[0].at[pl.ds(sid * PER_SUB + t * SCH, SCH)],
                bufs[t % 2], lsems[t % 2])
            for t in range(N_SCH)
        ]
        stores = [
            pltpu.make_async_copy(
                bufs[t % 2],
                w_sh.at[pl.ds(sid * PER_SUB + t * SCH, SCH)], ssems[t % 2])
            for t in range(N_SCH)
        ]
        loads[0].start()
        for t in range(N_SCH):
            if t + 1 < N_SCH:
                if t - 1 >= 0:
                    stores[t - 1].wait()
                loads[t + 1].start()
            loads[t].wait()
            stores[t].start()
        stores[N_SCH - 2].wait()
        stores[N_SCH - 1].wait()

        @pl.when(sid == 0)
        def _():
            pltpu.async_copy(w_hbm.at[0].at[pl.ds(NS * PER_SUB, REM)],
                             buf_a.at[pl.ds(0, REM)], la).wait()
            pltpu.async_copy(buf_a.at[pl.ds(0, REM)],
                             w_sh.at[pl.ds(NS * PER_SUB, REM)], sa).wait()

        for c in idx_copies:
            c.wait()
        plsc.subcore_barrier()
        F1 = N_FIELDS // 2
        G1 = F1 * B_PER_W
        g1 = pltpu.make_async_copy(
            w_sh.at[idx_v.at[pl.ds(0, G1)]], vals_v.at[pl.ds(0, G1)], sem)
        g2 = pltpu.make_async_copy(
            w_sh.at[idx_v.at[pl.ds(G1, IDX_PER_W - G1)]],
            vals_v.at[pl.ds(G1, IDX_PER_W - G1)], la)
        g1.start()
        g2.start()
        g1.wait()

        def reduce_pass1(jg, _):
            base = jg * LANES
            acc = vals_v[pl.ds(base, LANES)]
            for f in range(1, F1):
                acc = acc + vals_v[pl.ds(f * B_PER_W + base, LANES)]
            out_v[pl.ds(base, LANES)] = acc
            return 0

        lax.fori_loop(0, B_PER_W // LANES, reduce_pass1, 0, unroll=False)
        g2.wait()

        def reduce_pass2(jg, _):
            base = jg * LANES
            acc = out_v[pl.ds(base, LANES)]
            for f in range(F1, N_FIELDS):
                acc = acc + vals_v[pl.ds(f * B_PER_W + base, LANES)]
            out_v[pl.ds(base, LANES)] = acc
            return 0

        lax.fori_loop(0, B_PER_W // LANES, reduce_pass2, 0, unroll=False)
        pltpu.sync_copy(out_v, out_hbm.at[0].at[pl.ds(wid * B_PER_W, B_PER_W)])

    return k(idx_f, w_row)


def _second_order_tc(et):
    BLK = 2048

    def body(e_ref, out_ref):
        x = e_ref[...]
        s = jnp.sum(x, axis=0)
        t1 = jnp.sum(s * s, axis=0, keepdims=True)
        t2 = jnp.sum(jnp.sum(x * x, axis=0), axis=0, keepdims=True)
        out_ref[...] = 0.5 * (t1 - t2)

    return pl.pallas_call(
        body,
        grid=(BATCH // BLK,),
        in_specs=[pl.BlockSpec((N_FIELDS, EMBED_DIM, BLK), lambda i: (0, 0, i))],
        out_specs=pl.BlockSpec((1, BLK), lambda i: (0, i)),
        out_shape=jax.ShapeDtypeStruct((1, BATCH), jnp.float32),
    )(et)


def _add_tc(a, b):
    def body(a_ref, b_ref, out_ref):
        out_ref[...] = a_ref[...] + b_ref[...]

    return pl.pallas_call(
        body,
        out_shape=jax.ShapeDtypeStruct((1, BATCH), jnp.float32),
    )(a, b)


def kernel(sparse_inputs, embedding_inputs, w):
    idx_f = sparse_inputs.T.reshape(-1)
    first = _first_order_sc(idx_f, w.T)
    second = _second_order_tc(embedding_inputs.transpose(1, 2, 0))
    return _add_tc(first, second).T


# speedup vs baseline: 1.0050x; 1.0050x over previous
import functools

import jax
import jax.numpy as jnp
from jax import lax
from jax.experimental import pallas as pl
from jax.experimental.pallas import tpu as pltpu
from jax.experimental.pallas import tpu_sc as plsc

BATCH = 16384
N_FIELDS = 26
EMBED_DIM = 16

NC, NS, LANES = 2, 16, 16
NW = NC * NS
B_PER_W = BATCH // NW
IDX_PER_W = B_PER_W * N_FIELDS

FEAT = 1000000
PER_SUB = 62496
SCH = 5208
N_SCH = PER_SUB // SCH
N_BUF = 4
REM = FEAT - NS * PER_SUB


def _first_order_sc(idx_f, w_row):
    mesh = plsc.VectorSubcoreMesh(core_axis_name="c", subcore_axis_name="s")

    @functools.partial(
        pl.kernel,
        out_type=jax.ShapeDtypeStruct((1, BATCH), jnp.float32),
        mesh=mesh,
        scratch_types=[
            pltpu.VMEM((IDX_PER_W,), jnp.int32),
            pltpu.VMEM((IDX_PER_W,), jnp.float32),
            pltpu.VMEM((B_PER_W,), jnp.float32),
            pltpu.VMEM((SCH,), jnp.float32),
            pltpu.VMEM((SCH,), jnp.float32),
            pltpu.VMEM((SCH,), jnp.float32),
            pltpu.VMEM((SCH,), jnp.float32),
            pltpu.VMEM_SHARED((FEAT,), jnp.float32),
            pltpu.SemaphoreType.DMA,
            pltpu.SemaphoreType.DMA,
            pltpu.SemaphoreType.DMA,
            pltpu.SemaphoreType.DMA,
            pltpu.SemaphoreType.DMA,
            pltpu.SemaphoreType.DMA,
            pltpu.SemaphoreType.DMA,
            pltpu.SemaphoreType.DMA,
            pltpu.SemaphoreType.DMA,
            pltpu.SemaphoreType.DMA,
        ],
    )
    def k(idx_hbm, w_hbm, out_hbm, idx_v, vals_v, out_v,
          buf_a, buf_b, buf_c, buf_d,
          w_sh, sem, la, lb, lc, ld, sa, sb, sc_, sd, isem):
        sid = lax.axis_index("s")
        wid = sid * NC + lax.axis_index("c")
        idx_copies = [
            pltpu.make_async_copy(
                idx_hbm.at[pl.ds(f * BATCH + wid * B_PER_W, B_PER_W)],
                idx_v.at[pl.ds(f * B_PER_W, B_PER_W)], isem)
            for f in range(N_FIELDS)
        ]
        for c in idx_copies:
            c.start()

        bufs = (buf_a, buf_b, buf_c, buf_d)
        lsems = (la, lb, lc, ld)
        ssems = (sa, sb, sc_, sd)
        loads = [
            pltpu.make_async_copy(
                w_hbm.at[0].at[pl.ds(sid * PER_SUB + t * SCH, SCH)],
                bufs[t % N_BUF], lsems[t % N_BUF])
            for t in range(N_SCH)
        ]
        stores = [
            pltpu.make_async_copy(
                bufs[t % N_BUF],
                w_sh.at[pl.ds(sid * PER_SUB + t * SCH, SCH)], ssems[t % N_BUF])
            for t in range(N_SCH)
        ]
        LOOK = 2
        for t in range(LOOK):
            loads[t].start()
        for t in range(N_SCH):
            nxt = t + LOOK
            if nxt < N_SCH:
                if nxt - N_BUF >= 0:
                    stores[nxt - N_BUF].wait()
                loads[nxt].start()
            loads[t].wait()
            stores[t].start()
        for t in range(N_SCH - N_BUF, N_SCH):
            stores[t].wait()

        @pl.when(sid == 0)
        def _():
            pltpu.async_copy(w_hbm.at[0].at[pl.ds(NS * PER_SUB, REM)],
                             buf_a.at[pl.ds(0, REM)], la).wait()
            pltpu.async_copy(buf_a.at[pl.ds(0, REM)],
                             w_sh.at[pl.ds(NS * PER_SUB, REM)], sa).wait()

        for c in idx_copies:
            c.wait()
        plsc.subcore_barrier()
        F1 = N_FIELDS // 2
        G1 = F1 * B_PER_W
        g1 = pltpu.make_async_copy(
            w_sh.at[idx_v.at[pl.ds(0, G1)]], vals_v.at[pl.ds(0, G1)], sem)
        g2 = pltpu.make_async_copy(
            w_sh.at[idx_v.at[pl.ds(G1, IDX_PER_W - G1)]],
            vals_v.at[pl.ds(G1, IDX_PER_W - G1)], la)
        g1.start()
        g2.start()
        g1.wait()

        def reduce_pass1(jg, _):
            base = jg * LANES
            acc = vals_v[pl.ds(base, LANES)]
            for f in range(1, F1):
                acc = acc + vals_v[pl.ds(f * B_PER_W + base, LANES)]
            out_v[pl.ds(base, LANES)] = acc
            return 0

        lax.fori_loop(0, B_PER_W // LANES, reduce_pass1, 0, unroll=False)
        g2.wait()

        def reduce_pass2(jg, _):
            base = jg * LANES
            acc = out_v[pl.ds(base, LANES)]
            for f in range(F1, N_FIELDS):
                acc = acc + vals_v[pl.ds(f * B_PER_W + base, LANES)]
            out_v[pl.ds(base, LANES)] = acc
            return 0

        lax.fori_loop(0, B_PER_W // LANES, reduce_pass2, 0, unroll=False)
        pltpu.sync_copy(out_v, out_hbm.at[0].at[pl.ds(wid * B_PER_W, B_PER_W)])

    return k(idx_f, w_row)


def _second_order_tc(et):
    BLK = 2048

    def body(e_ref, out_ref):
        x = e_ref[...]
        s = jnp.sum(x, axis=0)
        t1 = jnp.sum(s * s, axis=0, keepdims=True)
        t2 = jnp.sum(jnp.sum(x * x, axis=0), axis=0, keepdims=True)
        out_ref[...] = 0.5 * (t1 - t2)

    return pl.pallas_call(
        body,
        grid=(BATCH // BLK,),
        in_specs=[pl.BlockSpec((N_FIELDS, EMBED_DIM, BLK), lambda i: (0, 0, i))],
        out_specs=pl.BlockSpec((1, BLK), lambda i: (0, i)),
        out_shape=jax.ShapeDtypeStruct((1, BATCH), jnp.float32),
    )(et)


def _add_tc(a, b):
    def body(a_ref, b_ref, out_ref):
        out_ref[...] = a_ref[...] + b_ref[...]

    return pl.pallas_call(
        body,
        out_shape=jax.ShapeDtypeStruct((1, BATCH), jnp.float32),
    )(a, b)


def kernel(sparse_inputs, embedding_inputs, w):
    idx_f = sparse_inputs.T.reshape(-1)
    first = _first_order_sc(idx_f, w.T)
    second = _second_order_tc(embedding_inputs.transpose(1, 2, 0))
    return _add_tc(first, second).T
